# Initial kernel scaffold; baseline (speedup 1.0000x reference)
import jax
import jax.numpy as jnp
from jax.experimental import pallas as pl


def kernel(x, edge_attr, u, We1, be1, We2, be2, Wn1, bn1, Wn2, bn2, Wg1, bg1, Wg2, bg2, lns_n, lnb_n, lns_e, lnb_e, lns_g, lnb_g, edge_index, batch):
    raise NotImplementedError("write your pallas kernel here")


# speedup vs baseline: 3.8621x; 3.8621x over previous
import functools

import jax
import jax.numpy as jnp
from jax import lax
from jax.experimental import pallas as pl
from jax.experimental.pallas import tpu as pltpu
from jax.experimental.pallas import tpu_sc as plsc

F32 = jnp.float32
_EPS = 1e-5

_NC = 2
_NS = 16
_NW = _NC * _NS


def _ln(v, scale, bias):
    mu = jnp.mean(v, axis=-1, keepdims=True)
    var = jnp.mean((v - mu) ** 2, axis=-1, keepdims=True)
    return (v - mu) / jnp.sqrt(var + _EPS) * scale + bias


def _proj_body(x_ref, we1_ref, u_ref, be1_ref, ps_ref, pd_ref, cu_ref):
    d = x_ref.shape[1]
    xv = x_ref[...]
    w = we1_ref[...]
    ps_ref[...] = jnp.dot(xv, w[d:2 * d, :], preferred_element_type=F32)
    pd_ref[...] = jnp.dot(xv, w[2 * d:3 * d, :], preferred_element_type=F32)

    @pl.when(pl.program_id(0) == 0)
    def _():
        cu_ref[...] = (
            jnp.dot(u_ref[...], w[3 * d:4 * d, :], preferred_element_type=F32)
            + be1_ref[...]
        )


def _projections(x, we1_l, u, be1_l, bn):
    n, d = x.shape
    grid = (n // bn,)
    return pl.pallas_call(
        _proj_body,
        grid=grid,
        in_specs=[
            pl.BlockSpec((bn, d), lambda i: (i, 0)),
            pl.BlockSpec((4 * d, d), lambda i: (0, 0)),
            pl.BlockSpec((1, d), lambda i: (0, 0)),
            pl.BlockSpec((1, d), lambda i: (0, 0)),
        ],
        out_specs=[
            pl.BlockSpec((bn, d), lambda i: (i, 0)),
            pl.BlockSpec((bn, d), lambda i: (i, 0)),
            pl.BlockSpec((1, d), lambda i: (0, 0)),
        ],
        out_shape=[
            jax.ShapeDtypeStruct((n, d), F32),
            jax.ShapeDtypeStruct((n, d), F32),
            jax.ShapeDtypeStruct((1, d), F32),
        ],
    )(x, we1_l, u, be1_l)


def _make_gather(n, d, e, ch):
    epw = e // _NW
    nch = epw // ch
    mesh = plsc.VectorSubcoreMesh(core_axis_name="c", subcore_axis_name="s")

    @functools.partial(
        pl.kernel,
        out_type=(
            jax.ShapeDtypeStruct((e, d), F32),
            jax.ShapeDtypeStruct((e, d), F32),
        ),
        mesh=mesh,
        scratch_types=[
            pltpu.VMEM((ch,), jnp.int32),
            pltpu.VMEM((ch,), jnp.int32),
            pltpu.VMEM((ch, d), F32),
            pltpu.VMEM((ch, d), F32),
            pltpu.SemaphoreType.DMA,
            pltpu.SemaphoreType.DMA,
        ],
    )
    def gather(ps_hbm, pd_hbm, si_hbm, di_hbm, g1_hbm, g2_hbm,
               iv1, iv2, ra, rb, s1, s2):
        wid = lax.axis_index("c") * _NS + lax.axis_index("s")
        base = pl.multiple_of(wid * epw, 8)

        def body(i, carry):
            off = pl.multiple_of(base + i * ch, 8)
            pltpu.sync_copy(si_hbm.at[pl.ds(off, ch)], iv1)
            pltpu.sync_copy(di_hbm.at[pl.ds(off, ch)], iv2)
            cp1 = pltpu.async_copy(ps_hbm.at[iv1], ra, s1)
            cp2 = pltpu.async_copy(pd_hbm.at[iv2], rb, s2)
            cp1.wait()
            cp2.wait()
            pltpu.sync_copy(ra, g1_hbm.at[pl.ds(off, ch)])
            pltpu.sync_copy(rb, g2_hbm.at[pl.ds(off, ch)])
            return carry

        lax.fori_loop(0, nch, body, 0)

    return gather


def _make_scatter(n, d, e, ch):
    epw = e // _NW
    nch = epw // ch
    drain_a = (n // _NS) & ~7
    drain_last = n - drain_a * (_NS - 1)
    mesh = plsc.VectorSubcoreMesh(core_axis_name="c", subcore_axis_name="s")

    @functools.partial(
        pl.kernel,
        out_type=jax.ShapeDtypeStruct((_NC * n, d), F32),
        mesh=mesh,
        scratch_types=[
            pltpu.VMEM((ch,), jnp.int32),
            pltpu.VMEM((ch, d), F32),
            pltpu.VMEM_SHARED((n, d), F32),
        ],
    )
    def scatter(e_hbm, di_hbm, zer_hbm, out_hbm, iv, rows, acc_sh):
        cid = lax.axis_index("c")
        sid = lax.axis_index("s")
        wid = cid * _NS + sid
        base = pl.multiple_of(wid * epw, 8)

        @pl.when(sid == 0)
        def _():
            pltpu.sync_copy(zer_hbm, acc_sh)

        plsc.subcore_barrier()

        def body(i, carry):
            off = pl.multiple_of(base + i * ch, 8)
            pltpu.sync_copy(di_hbm.at[pl.ds(off, ch)], iv)
            pltpu.sync_copy(e_hbm.at[pl.ds(off, ch)], rows)
            pltpu.sync_copy(rows, acc_sh.at[iv], add=True)
            return carry

        lax.fori_loop(0, nch, body, 0)
        plsc.subcore_barrier()

        @pl.when(sid < _NS - 1)
        def _():
            r0 = pl.multiple_of(sid * drain_a, 8)
            pltpu.sync_copy(
                acc_sh.at[pl.ds(r0, drain_a)],
                out_hbm.at[pl.ds(cid * n + r0, drain_a)],
            )

        @pl.when(sid == _NS - 1)
        def _():
            r0 = pl.multiple_of((_NS - 1) * drain_a, 8)
            pltpu.sync_copy(
                acc_sh.at[pl.ds(r0, drain_last)],
                out_hbm.at[pl.ds(cid * n + r0, drain_last)],
            )

    return scatter


def _edge_body(e_ref, g1_ref, g2_ref, a_ref, w2_ref, cu_ref, be2_ref,
               lns_ref, lnb_ref, enew_ref, efin_ref, esum_ref):
    i = pl.program_id(0)

    @pl.when(i == 0)
    def _():
        esum_ref[...] = jnp.zeros_like(esum_ref)

    e0 = e_ref[...]
    h = (
        jnp.dot(e0, a_ref[...], preferred_element_type=F32)
        + g1_ref[...]
        + g2_ref[...]
        + cu_ref[...]
    )
    h = jnp.maximum(h, 0.0)
    en = jnp.dot(h, w2_ref[...], preferred_element_type=F32) + be2_ref[...]
    enew_ref[...] = en
    esum_ref[...] += jnp.sum(en, axis=0, keepdims=True)
    efin_ref[...] = _ln(en + e0, lns_ref[...], lnb_ref[...])


def _edge_mlp(e, g1, g2, a, w2, cu, be2, lns, lnb, be):
    ne, d = e.shape
    grid = (ne // be,)
    blk = pl.BlockSpec((be, d), lambda i: (i, 0))
    full = pl.BlockSpec((d, d), lambda i: (0, 0))
    row = pl.BlockSpec((1, d), lambda i: (0, 0))
    return pl.pallas_call(
        _edge_body,
        grid=grid,
        in_specs=[blk, blk, blk, full, full, row, row, row, row],
        out_specs=[blk, blk, row],
        out_shape=[
            jax.ShapeDtypeStruct((ne, d), F32),
            jax.ShapeDtypeStruct((ne, d), F32),
            jax.ShapeDtypeStruct((1, d), F32),
        ],
    )(e, g1, g2, a, w2, cu, be2, lns, lnb)


def _node_body(nblocks, n_nodes, n_edges,
               x_ref, p0_ref, p1_ref, u_ref, wn1_ref, bn1_ref, wn2_ref,
               bn2_ref, lnsn_ref, lnbn_ref, esum_ref, wg1_ref, bg1_ref,
               wg2_ref, bg2_ref, lnsg_ref, lnbg_ref,
               xfin_ref, ufin_ref, xsum_ref):
    i = pl.program_id(0)
    d = x_ref.shape[1]

    @pl.when(i == 0)
    def _():
        xsum_ref[...] = jnp.zeros_like(xsum_ref)
        ufin_ref[...] = jnp.zeros_like(ufin_ref)

    x0 = x_ref[...]
    agg = p0_ref[...] + p1_ref[...]
    wn1 = wn1_ref[...]
    u0 = u_ref[...]
    cu = jnp.dot(u0, wn1[2 * d:3 * d, :], preferred_element_type=F32) + bn1_ref[...]
    h = jnp.maximum(
        jnp.dot(x0, wn1[:d, :], preferred_element_type=F32)
        + jnp.dot(agg, wn1[d:2 * d, :], preferred_element_type=F32)
        + cu,
        0.0,
    )
    xn = jnp.dot(h, wn2_ref[...], preferred_element_type=F32) + bn2_ref[...]
    xsum_ref[...] += jnp.sum(xn, axis=0, keepdims=True)
    xfin_ref[...] = _ln(xn + x0, lnsn_ref[...], lnbn_ref[...])

    @pl.when(i == nblocks - 1)
    def _():
        node_mean = xsum_ref[...] * (1.0 / n_nodes)
        edge_mean = esum_ref[...] * (1.0 / n_edges)
        wg1 = wg1_ref[...]
        gi = (
            jnp.dot(u0, wg1[:d, :], preferred_element_type=F32)
            + jnp.dot(node_mean, wg1[d:2 * d, :], preferred_element_type=F32)
            + jnp.dot(edge_mean, wg1[2 * d:3 * d, :], preferred_element_type=F32)
            + bg1_ref[...]
        )
        hg = jnp.maximum(gi, 0.0)
        un = jnp.dot(hg, wg2_ref[...], preferred_element_type=F32) + bg2_ref[...]
        ufin_ref[...] = _ln(un + u0, lnsg_ref[...], lnbg_ref[...])


def _node_mlp(x, p0, p1, u, wn1, bn1, wn2, bn2, lnsn, lnbn,
              esum, wg1, bg1, wg2, bg2, lnsg, lnbg, bn, n_edges):
    n, d = x.shape
    nblocks = n // bn
    blk = pl.BlockSpec((bn, d), lambda i: (i, 0))
    full2 = pl.BlockSpec((2 * d, d), lambda i: (0, 0))
    full3 = pl.BlockSpec((3 * d, d), lambda i: (0, 0))
    fulld = pl.BlockSpec((d, d), lambda i: (0, 0))
    row = pl.BlockSpec((1, d), lambda i: (0, 0))
    body = functools.partial(_node_body, nblocks, float(n), float(n_edges))
    xfin, ufin, _ = pl.pallas_call(
        body,
        grid=(nblocks,),
        in_specs=[blk, blk, blk, row, full3, row, fulld, row, row, row,
                  row, full3, row, fulld, row, row, row],
        out_specs=[blk, row, row],
        out_shape=[
            jax.ShapeDtypeStruct((n, d), F32),
            jax.ShapeDtypeStruct((1, d), F32),
            jax.ShapeDtypeStruct((1, d), F32),
        ],
    )(x, p0, p1, u, wn1, bn1, wn2, bn2, lnsn, lnbn,
      esum, wg1, bg1, wg2, bg2, lnsg, lnbg)
    return xfin, ufin


def kernel(x, edge_attr, u, We1, be1, We2, be2, Wn1, bn1, Wn2, bn2,
           Wg1, bg1, Wg2, bg2, lns_n, lnb_n, lns_e, lnb_e, lns_g, lnb_g,
           edge_index, batch):
    n, d = x.shape
    e_cnt = edge_attr.shape[0]
    n_layers = We1.shape[0]

    src = edge_index[0]
    dst = edge_index[1]

    ch = 80
    bn_node = 2000
    be_edge = 3200

    gather = _make_gather(n, d, e_cnt, ch)
    scatter = _make_scatter(n, d, e_cnt, ch)
    zeros_nd = jnp.zeros((n, d), F32)

    def row(v):
        return v.reshape(1, d)

    xc, ec, uc = x, edge_attr, u.reshape(1, d)
    for l in range(n_layers):
        we1_l = We1[l]
        ps, pd, cu = _projections(xc, we1_l, uc, row(be1[l]), bn_node)
        g1, g2 = gather(ps, pd, src, dst)
        e_new, e_fin, e_sum = _edge_mlp(
            ec, g1, g2, we1_l[:d, :], We2[l], cu, row(be2[l]),
            row(lns_e[l]), row(lnb_e[l]), be_edge)
        parts = scatter(e_new, dst, zeros_nd)
        x_fin, u_fin = _node_mlp(
            xc, parts[:n, :], parts[n:, :], uc,
            Wn1[l], row(bn1[l]), Wn2[l], row(bn2[l]),
            row(lns_n[l]), row(lnb_n[l]),
            e_sum, Wg1[l], row(bg1[l]), Wg2[l], row(bg2[l]),
            row(lns_g[l]), row(lnb_g[l]),
            bn_node, e_cnt)
        xc, ec, uc = x_fin, e_fin, u_fin

    return xc, ec, uc.reshape(u.shape)


# speedup vs baseline: 5.6174x; 1.4545x over previous
import functools

import jax
import jax.numpy as jnp
from jax import lax
from jax.experimental import pallas as pl
from jax.experimental.pallas import tpu as pltpu
from jax.experimental.pallas import tpu_sc as plsc

F32 = jnp.float32
_EPS = 1e-5

_NC = 2
_NS = 16
_NW = _NC * _NS


def _ln(v, scale, bias):
    mu = jnp.mean(v, axis=-1, keepdims=True)
    var = jnp.mean((v - mu) ** 2, axis=-1, keepdims=True)
    return (v - mu) / jnp.sqrt(var + _EPS) * scale + bias


def _proj_body(x_ref, we1_ref, u_ref, be1_ref, ps_ref, pd_ref, cu_ref):
    d = x_ref.shape[1]
    xv = x_ref[...]
    w = we1_ref[...]
    ps_ref[...] = jnp.dot(xv, w[d:2 * d, :], preferred_element_type=F32)
    pd_ref[...] = jnp.dot(xv, w[2 * d:3 * d, :], preferred_element_type=F32)

    @pl.when(pl.program_id(0) == 0)
    def _():
        cu_ref[...] = (
            jnp.dot(u_ref[...], w[3 * d:4 * d, :], preferred_element_type=F32)
            + be1_ref[...]
        )


def _projections(x, we1_l, u, be1_l, bn):
    n, d = x.shape
    grid = (n // bn,)
    return pl.pallas_call(
        _proj_body,
        grid=grid,
        in_specs=[
            pl.BlockSpec((bn, d), lambda i: (i, 0)),
            pl.BlockSpec((4 * d, d), lambda i: (0, 0)),
            pl.BlockSpec((1, d), lambda i: (0, 0)),
            pl.BlockSpec((1, d), lambda i: (0, 0)),
        ],
        out_specs=[
            pl.BlockSpec((bn, d), lambda i: (i, 0)),
            pl.BlockSpec((bn, d), lambda i: (i, 0)),
            pl.BlockSpec((1, d), lambda i: (0, 0)),
        ],
        out_shape=[
            jax.ShapeDtypeStruct((n, d), F32),
            jax.ShapeDtypeStruct((n, d), F32),
            jax.ShapeDtypeStruct((1, d), F32),
        ],
    )(x, we1_l, u, be1_l)


_NBUF = 5


def _make_gather(n, d, e, ch):
    epw = e // _NW
    nch = epw // ch
    nb = _NBUF
    nk = nch // nb
    assert nch == nb * nk
    mesh = plsc.VectorSubcoreMesh(core_axis_name="c", subcore_axis_name="s")

    @functools.partial(
        pl.kernel,
        out_type=(
            jax.ShapeDtypeStruct((e, d), F32),
            jax.ShapeDtypeStruct((e, d), F32),
        ),
        mesh=mesh,
        scratch_types=(
            [pltpu.VMEM((ch,), jnp.int32) for _ in range(2 * nb)]
            + [
                pltpu.VMEM((nb, ch, d), F32),
                pltpu.VMEM((nb, ch, d), F32),
                pltpu.SemaphoreType.DMA((nb,)),
                pltpu.SemaphoreType.DMA((nb,)),
                pltpu.SemaphoreType.DMA((nb,)),
            ]
        ),
    )
    def gather(ps_hbm, pd_hbm, si_hbm, di_hbm, g1_hbm, g2_hbm,
               is0, is1, is2, is3, is4, id0, id1, id2, id3, id4,
               ra, rb, i_sem, g_sem, w_sem):
        ivs = [is0, is1, is2, is3, is4]
        ivd = [id0, id1, id2, id3, id4]
        wid = lax.axis_index("c") * _NS + lax.axis_index("s")
        base = pl.multiple_of(wid * epw, 8)

        def issue_idx(c, b):
            off = pl.multiple_of(base + c * ch, 8)
            pltpu.async_copy(si_hbm.at[pl.ds(off, ch)], ivs[b], i_sem.at[b])
            pltpu.async_copy(di_hbm.at[pl.ds(off, ch)], ivd[b], i_sem.at[b])

        def wait_idx(b):
            pltpu.make_async_copy(si_hbm.at[pl.ds(0, ch)], ivs[b],
                                  i_sem.at[b]).wait()
            pltpu.make_async_copy(di_hbm.at[pl.ds(0, ch)], ivd[b],
                                  i_sem.at[b]).wait()

        def issue_gather(c, b):
            pltpu.async_copy(ps_hbm.at[ivs[b]], ra.at[b], g_sem.at[b])
            pltpu.async_copy(pd_hbm.at[ivd[b]], rb.at[b], g_sem.at[b])

        def wait_gather(b):
            pltpu.make_async_copy(ps_hbm.at[pl.ds(0, ch)], ra.at[b],
                                  g_sem.at[b]).wait()
            pltpu.make_async_copy(pd_hbm.at[pl.ds(0, ch)], rb.at[b],
                                  g_sem.at[b]).wait()

        def issue_write(c, b):
            off = pl.multiple_of(base + c * ch, 8)
            pltpu.async_copy(ra.at[b], g1_hbm.at[pl.ds(off, ch)], w_sem.at[b])
            pltpu.async_copy(rb.at[b], g2_hbm.at[pl.ds(off, ch)], w_sem.at[b])

        def wait_write(b):
            pltpu.make_async_copy(ra.at[b], g1_hbm.at[pl.ds(0, ch)],
                                  w_sem.at[b]).wait()
            pltpu.make_async_copy(rb.at[b], g2_hbm.at[pl.ds(0, ch)],
                                  w_sem.at[b]).wait()

        for b in range(nb):
            issue_idx(b, b)
        for b in range(nb):
            wait_idx(b)
            issue_gather(b, b)
            if b >= 1:
                wait_gather(b - 1)
                issue_write(b - 1, b - 1)
        issue_idx(nb, 0)

        def body(k, carry):
            for b in range(nb):
                c = k * nb + b
                wait_write(b)
                wait_idx(b)
                issue_gather(c, b)
                issue_idx(c + 1, (b + 1) % nb)
                b2 = (b - 1) % nb
                wait_gather(b2)
                issue_write(c - 1, b2)
            return carry

        lax.fori_loop(1, nk - 1, body, 0)

        for b in range(nb):
            c = (nk - 1) * nb + b
            wait_write(b)
            wait_idx(b)
            issue_gather(c, b)
            if b < nb - 1:
                issue_idx(c + 1, b + 1)
            b2 = (b - 1) % nb
            wait_gather(b2)
            issue_write(c - 1, b2)

        wait_gather(nb - 1)
        issue_write(nch - 1, nb - 1)
        for b in range(nb):
            wait_write(b)

    return gather


def _make_scatter(n, d, e, ch):
    epw = e // _NW
    nch = epw // ch
    nb = _NBUF
    nk = nch // nb
    assert nch == nb * nk
    drain_a = (n // _NS) & ~7
    drain_last = n - drain_a * (_NS - 1)
    mesh = plsc.VectorSubcoreMesh(core_axis_name="c", subcore_axis_name="s")

    @functools.partial(
        pl.kernel,
        out_type=jax.ShapeDtypeStruct((_NC * n, d), F32),
        mesh=mesh,
        scratch_types=(
            [pltpu.VMEM((ch,), jnp.int32) for _ in range(nb)]
            + [
                pltpu.VMEM((nb, ch, d), F32),
                pltpu.VMEM_SHARED((n, d), F32),
                pltpu.SemaphoreType.DMA((nb,)),
                pltpu.SemaphoreType.DMA((nb,)),
            ]
        ),
    )
    def scatter(e_hbm, di_hbm, zer_hbm, out_hbm, iv0, iv1, iv2, iv3, iv4,
                rows, acc_sh, r_sem, sc_sem):
        ivb = [iv0, iv1, iv2, iv3, iv4]
        cid = lax.axis_index("c")
        sid = lax.axis_index("s")
        wid = cid * _NS + sid
        base = pl.multiple_of(wid * epw, 8)

        r0a = pl.multiple_of(sid * drain_a, 8)
        r0l = pl.multiple_of((_NS - 1) * drain_a, 8)

        @pl.when(sid < _NS - 1)
        def _():
            pltpu.sync_copy(zer_hbm.at[pl.ds(r0a, drain_a)],
                            acc_sh.at[pl.ds(r0a, drain_a)])

        @pl.when(sid == _NS - 1)
        def _():
            pltpu.sync_copy(zer_hbm.at[pl.ds(r0l, drain_last)],
                            acc_sh.at[pl.ds(r0l, drain_last)])

        plsc.subcore_barrier()

        def issue_load(c, b):
            off = pl.multiple_of(base + c * ch, 8)
            pltpu.async_copy(e_hbm.at[pl.ds(off, ch)], rows.at[b],
                             r_sem.at[b])
            pltpu.async_copy(di_hbm.at[pl.ds(off, ch)], ivb[b], r_sem.at[b])

        def wait_load(b):
            pltpu.make_async_copy(e_hbm.at[pl.ds(0, ch)], rows.at[b],
                                  r_sem.at[b]).wait()
            pltpu.make_async_copy(di_hbm.at[pl.ds(0, ch)], ivb[b],
                                  r_sem.at[b]).wait()

        def issue_scatter(c, b):
            pltpu.async_copy(rows.at[b], acc_sh.at[ivb[b]], sc_sem.at[b],
                             add=True)

        def wait_scatter(b):
            pltpu.make_async_copy(rows.at[b], acc_sh.at[pl.ds(0, ch)],
                                  sc_sem.at[b]).wait()

        for b in range(nb):
            issue_load(b, b)
        for b in range(1, nb):
            wait_load(b - 1)
            issue_scatter(b - 1, b - 1)

        def body(k, carry):
            for b in range(nb):
                c = k * nb + b
                wait_scatter(b)
                issue_load(c, b)
                b2 = (b - 1) % nb
                wait_load(b2)
                issue_scatter(c - 1, b2)
            return carry

        lax.fori_loop(1, nk, body, 0)

        wait_load(nb - 1)
        issue_scatter(nch - 1, nb - 1)
        for b in range(nb):
            wait_scatter(b)

        plsc.subcore_barrier()

        @pl.when(sid < _NS - 1)
        def _():
            pltpu.sync_copy(acc_sh.at[pl.ds(r0a, drain_a)],
                            out_hbm.at[pl.ds(cid * n + r0a, drain_a)])

        @pl.when(sid == _NS - 1)
        def _():
            pltpu.sync_copy(acc_sh.at[pl.ds(r0l, drain_last)],
                            out_hbm.at[pl.ds(cid * n + r0l, drain_last)])

    return scatter


def _edge_body(e_ref, g1_ref, g2_ref, a_ref, w2_ref, cu_ref, be2_ref,
               lns_ref, lnb_ref, enew_ref, efin_ref, esum_ref):
    i = pl.program_id(0)

    @pl.when(i == 0)
    def _():
        esum_ref[...] = jnp.zeros_like(esum_ref)

    e0 = e_ref[...]
    h = (
        jnp.dot(e0, a_ref[...], preferred_element_type=F32)
        + g1_ref[...]
        + g2_ref[...]
        + cu_ref[...]
    )
    h = jnp.maximum(h, 0.0)
    en = jnp.dot(h, w2_ref[...], preferred_element_type=F32) + be2_ref[...]
    enew_ref[...] = en
    esum_ref[...] += jnp.sum(en, axis=0, keepdims=True)
    efin_ref[...] = _ln(en + e0, lns_ref[...], lnb_ref[...])


def _edge_mlp(e, g1, g2, a, w2, cu, be2, lns, lnb, be):
    ne, d = e.shape
    grid = (ne // be,)
    blk = pl.BlockSpec((be, d), lambda i: (i, 0))
    full = pl.BlockSpec((d, d), lambda i: (0, 0))
    row = pl.BlockSpec((1, d), lambda i: (0, 0))
    return pl.pallas_call(
        _edge_body,
        grid=grid,
        in_specs=[blk, blk, blk, full, full, row, row, row, row],
        out_specs=[blk, blk, row],
        out_shape=[
            jax.ShapeDtypeStruct((ne, d), F32),
            jax.ShapeDtypeStruct((ne, d), F32),
            jax.ShapeDtypeStruct((1, d), F32),
        ],
    )(e, g1, g2, a, w2, cu, be2, lns, lnb)


def _node_body(nblocks, n_nodes, n_edges,
               x_ref, p0_ref, p1_ref, u_ref, wn1_ref, bn1_ref, wn2_ref,
               bn2_ref, lnsn_ref, lnbn_ref, esum_ref, wg1_ref, bg1_ref,
               wg2_ref, bg2_ref, lnsg_ref, lnbg_ref,
               xfin_ref, ufin_ref, xsum_ref):
    i = pl.program_id(0)
    d = x_ref.shape[1]

    @pl.when(i == 0)
    def _():
        xsum_ref[...] = jnp.zeros_like(xsum_ref)
        ufin_ref[...] = jnp.zeros_like(ufin_ref)

    x0 = x_ref[...]
    agg = p0_ref[...] + p1_ref[...]
    wn1 = wn1_ref[...]
    u0 = u_ref[...]
    cu = jnp.dot(u0, wn1[2 * d:3 * d, :], preferred_element_type=F32) + bn1_ref[...]
    h = jnp.maximum(
        jnp.dot(x0, wn1[:d, :], preferred_element_type=F32)
        + jnp.dot(agg, wn1[d:2 * d, :], preferred_element_type=F32)
        + cu,
        0.0,
    )
    xn = jnp.dot(h, wn2_ref[...], preferred_element_type=F32) + bn2_ref[...]
    xsum_ref[...] += jnp.sum(xn, axis=0, keepdims=True)
    xfin_ref[...] = _ln(xn + x0, lnsn_ref[...], lnbn_ref[...])

    @pl.when(i == nblocks - 1)
    def _():
        node_mean = xsum_ref[...] * (1.0 / n_nodes)
        edge_mean = esum_ref[...] * (1.0 / n_edges)
        wg1 = wg1_ref[...]
        gi = (
            jnp.dot(u0, wg1[:d, :], preferred_element_type=F32)
            + jnp.dot(node_mean, wg1[d:2 * d, :], preferred_element_type=F32)
            + jnp.dot(edge_mean, wg1[2 * d:3 * d, :], preferred_element_type=F32)
            + bg1_ref[...]
        )
        hg = jnp.maximum(gi, 0.0)
        un = jnp.dot(hg, wg2_ref[...], preferred_element_type=F32) + bg2_ref[...]
        ufin_ref[...] = _ln(un + u0, lnsg_ref[...], lnbg_ref[...])


def _node_mlp(x, p0, p1, u, wn1, bn1, wn2, bn2, lnsn, lnbn,
              esum, wg1, bg1, wg2, bg2, lnsg, lnbg, bn, n_edges):
    n, d = x.shape
    nblocks = n // bn
    blk = pl.BlockSpec((bn, d), lambda i: (i, 0))
    full2 = pl.BlockSpec((2 * d, d), lambda i: (0, 0))
    full3 = pl.BlockSpec((3 * d, d), lambda i: (0, 0))
    fulld = pl.BlockSpec((d, d), lambda i: (0, 0))
    row = pl.BlockSpec((1, d), lambda i: (0, 0))
    body = functools.partial(_node_body, nblocks, float(n), float(n_edges))
    xfin, ufin, _ = pl.pallas_call(
        body,
        grid=(nblocks,),
        in_specs=[blk, blk, blk, row, full3, row, fulld, row, row, row,
                  row, full3, row, fulld, row, row, row],
        out_specs=[blk, row, row],
        out_shape=[
            jax.ShapeDtypeStruct((n, d), F32),
            jax.ShapeDtypeStruct((1, d), F32),
            jax.ShapeDtypeStruct((1, d), F32),
        ],
    )(x, p0, p1, u, wn1, bn1, wn2, bn2, lnsn, lnbn,
      esum, wg1, bg1, wg2, bg2, lnsg, lnbg)
    return xfin, ufin


def kernel(x, edge_attr, u, We1, be1, We2, be2, Wn1, bn1, Wn2, bn2,
           Wg1, bg1, Wg2, bg2, lns_n, lnb_n, lns_e, lnb_e, lns_g, lnb_g,
           edge_index, batch):
    n, d = x.shape
    e_cnt = edge_attr.shape[0]
    n_layers = We1.shape[0]

    src = edge_index[0]
    dst = edge_index[1]

    ch_g = 80
    ch_s = 40
    bn_node = 2000
    be_edge = 3200

    gather = _make_gather(n, d, e_cnt, ch_g)
    scatter = _make_scatter(n, d, e_cnt, ch_s)
    zeros_nd = jnp.zeros((n, d), F32)

    def row(v):
        return v.reshape(1, d)

    xc, ec, uc = x, edge_attr, u.reshape(1, d)
    for l in range(n_layers):
        we1_l = We1[l]
        ps, pd, cu = _projections(xc, we1_l, uc, row(be1[l]), bn_node)
        g1, g2 = gather(ps, pd, src, dst)
        e_new, e_fin, e_sum = _edge_mlp(
            ec, g1, g2, we1_l[:d, :], We2[l], cu, row(be2[l]),
            row(lns_e[l]), row(lnb_e[l]), be_edge)
        parts = scatter(e_new, dst, zeros_nd)
        x_fin, u_fin = _node_mlp(
            xc, parts[:n, :], parts[n:, :], uc,
            Wn1[l], row(bn1[l]), Wn2[l], row(bn2[l]),
            row(lns_n[l]), row(lnb_n[l]),
            e_sum, Wg1[l], row(bg1[l]), Wg2[l], row(bg2[l]),
            row(lns_g[l]), row(lnb_g[l]),
            bn_node, e_cnt)
        xc, ec, uc = x_fin, e_fin, u_fin

    return xc, ec, uc.reshape(u.shape)


# speedup vs baseline: 6.3775x; 1.1353x over previous
import functools

import jax
import jax.numpy as jnp
from jax import lax
from jax.experimental import pallas as pl
from jax.experimental.pallas import tpu as pltpu
from jax.experimental.pallas import tpu_sc as plsc

F32 = jnp.float32
BF16 = jnp.bfloat16
_EPS = 1e-5

_NC = 2
_NS = 16
_NW = _NC * _NS


def _ln(v, scale, bias):
    mu = jnp.mean(v, axis=-1, keepdims=True)
    var = jnp.mean((v - mu) ** 2, axis=-1, keepdims=True)
    return (v - mu) / jnp.sqrt(var + _EPS) * scale + bias


def _proj_body(x_ref, we1_ref, u_ref, be1_ref, ps_ref, pd_ref, cu_ref):
    d = x_ref.shape[1]
    xv = x_ref[...]
    w = we1_ref[...]
    ps_ref[...] = jnp.dot(xv, w[d:2 * d, :], preferred_element_type=F32)
    pd_ref[...] = jnp.dot(xv, w[2 * d:3 * d, :], preferred_element_type=F32)

    @pl.when(pl.program_id(0) == 0)
    def _():
        cu_ref[...] = (
            jnp.dot(u_ref[...], w[3 * d:4 * d, :], preferred_element_type=F32)
            + be1_ref[...]
        )


def _projections(x, we1_l, u, be1_l, bn):
    n, d = x.shape
    grid = (n // bn,)
    return pl.pallas_call(
        _proj_body,
        grid=grid,
        in_specs=[
            pl.BlockSpec((bn, d), lambda i: (i, 0)),
            pl.BlockSpec((4 * d, d), lambda i: (0, 0)),
            pl.BlockSpec((1, d), lambda i: (0, 0)),
            pl.BlockSpec((1, d), lambda i: (0, 0)),
        ],
        out_specs=[
            pl.BlockSpec((bn, d), lambda i: (i, 0)),
            pl.BlockSpec((bn, d), lambda i: (i, 0)),
            pl.BlockSpec((1, d), lambda i: (0, 0)),
        ],
        out_shape=[
            jax.ShapeDtypeStruct((n, d), F32),
            jax.ShapeDtypeStruct((n, d), F32),
            jax.ShapeDtypeStruct((1, d), F32),
        ],
    )(x, we1_l, u, be1_l)


_NBUF = 5


def _make_gather(n, d, e, ch):
    epc = e // _NS
    nch = epc // ch
    nb = _NBUF
    nk = nch // nb
    assert nch == nb * nk
    lda = (n // _NS) & ~7
    ldl = n - lda * (_NS - 1)
    mesh = plsc.VectorSubcoreMesh(core_axis_name="c", subcore_axis_name="s")

    @functools.partial(
        pl.kernel,
        out_type=(
            jax.ShapeDtypeStruct((e, d), F32),
            jax.ShapeDtypeStruct((e, d), F32),
        ),
        mesh=mesh,
        scratch_types=(
            [pltpu.VMEM((ch,), jnp.int32) for _ in range(nb)]
            + [
                pltpu.VMEM((nb, ch, d), F32),
                pltpu.VMEM_SHARED((n, d), F32),
                pltpu.SemaphoreType.DMA((nb,)),
                pltpu.SemaphoreType.DMA((nb,)),
            ]
        ),
    )
    def gather(ps_hbm, pd_hbm, si_hbm, di_hbm, g1_hbm, g2_hbm,
               iv0, iv1, iv2, iv3, iv4, rows, tab_sh, i_sem, w_sem):
        ivb = [iv0, iv1, iv2, iv3, iv4]
        cid = lax.axis_index("c")
        sid = lax.axis_index("s")
        base = pl.multiple_of(sid * epc, 8)
        r0a = pl.multiple_of(sid * lda, 8)
        r0l = pl.multiple_of((_NS - 1) * lda, 8)

        def load_table(tab_hbm):
            @pl.when(sid < _NS - 1)
            def _():
                pltpu.sync_copy(tab_hbm.at[pl.ds(r0a, lda)],
                                tab_sh.at[pl.ds(r0a, lda)])

            @pl.when(sid == _NS - 1)
            def _():
                pltpu.sync_copy(tab_hbm.at[pl.ds(r0l, ldl)],
                                tab_sh.at[pl.ds(r0l, ldl)])

        @pl.when(cid == 0)
        def _():
            load_table(ps_hbm)

        @pl.when(cid == 1)
        def _():
            load_table(pd_hbm)

        plsc.subcore_barrier()

        def pipeline(ix_hbm, out_hbm):
            def issue_idx(c, b):
                off = pl.multiple_of(base + c * ch, 8)
                pltpu.async_copy(ix_hbm.at[pl.ds(off, ch)], ivb[b],
                                 i_sem.at[b])

            def wait_idx(b):
                pltpu.make_async_copy(ix_hbm.at[pl.ds(0, ch)], ivb[b],
                                      i_sem.at[b]).wait()

            def issue_write(c, b):
                off = pl.multiple_of(base + c * ch, 8)
                pltpu.async_copy(rows.at[b], out_hbm.at[pl.ds(off, ch)],
                                 w_sem.at[b])

            def wait_write(b):
                pltpu.make_async_copy(rows.at[b], out_hbm.at[pl.ds(0, ch)],
                                      w_sem.at[b]).wait()

            for b in range(nb):
                issue_idx(b, b)
            for b in range(nb):
                wait_idx(b)
                pltpu.sync_copy(tab_sh.at[ivb[b]], rows.at[b])
                issue_write(b, b)
                issue_idx(b + nb, b)

            def body(k, carry):
                for b in range(nb):
                    c = k * nb + b
                    wait_write(b)
                    wait_idx(b)
                    pltpu.sync_copy(tab_sh.at[ivb[b]], rows.at[b])
                    issue_write(c, b)

                    @pl.when(c + nb < nch)
                    def _():
                        issue_idx(c + nb, b)
                return carry

            lax.fori_loop(1, nk, body, 0)
            for b in range(nb):
                wait_write(b)

        @pl.when(cid == 0)
        def _():
            pipeline(si_hbm, g1_hbm)

        @pl.when(cid == 1)
        def _():
            pipeline(di_hbm, g2_hbm)

    return gather


def _make_scatter(n, d, e, ch):
    epw = e // _NW
    nch = epw // ch
    nb = _NBUF
    nk = nch // nb
    assert nch == nb * nk
    drain_a = (n // _NS) & ~7
    drain_last = n - drain_a * (_NS - 1)
    mesh = plsc.VectorSubcoreMesh(core_axis_name="c", subcore_axis_name="s")

    @functools.partial(
        pl.kernel,
        out_type=jax.ShapeDtypeStruct((_NC * n, d), F32),
        mesh=mesh,
        scratch_types=(
            [pltpu.VMEM((ch,), jnp.int32) for _ in range(nb)]
            + [
                pltpu.VMEM((nb, ch, d), F32),
                pltpu.VMEM_SHARED((n, d), F32),
                pltpu.SemaphoreType.DMA((nb,)),
                pltpu.SemaphoreType.DMA((nb,)),
            ]
        ),
    )
    def scatter(e_hbm, di_hbm, zer_hbm, out_hbm, iv0, iv1, iv2, iv3, iv4,
                rows, acc_sh, r_sem, sc_sem):
        ivb = [iv0, iv1, iv2, iv3, iv4]
        cid = lax.axis_index("c")
        sid = lax.axis_index("s")
        wid = cid * _NS + sid
        base = pl.multiple_of(wid * epw, 8)

        r0a = pl.multiple_of(sid * drain_a, 8)
        r0l = pl.multiple_of((_NS - 1) * drain_a, 8)

        @pl.when(sid < _NS - 1)
        def _():
            pltpu.sync_copy(zer_hbm.at[pl.ds(r0a, drain_a)],
                            acc_sh.at[pl.ds(r0a, drain_a)])

        @pl.when(sid == _NS - 1)
        def _():
            pltpu.sync_copy(zer_hbm.at[pl.ds(r0l, drain_last)],
                            acc_sh.at[pl.ds(r0l, drain_last)])

        plsc.subcore_barrier()

        def issue_load(c, b):
            off = pl.multiple_of(base + c * ch, 8)
            pltpu.async_copy(e_hbm.at[pl.ds(off, ch)], rows.at[b],
                             r_sem.at[b])
            pltpu.async_copy(di_hbm.at[pl.ds(off, ch)], ivb[b], r_sem.at[b])

        def wait_load(b):
            pltpu.make_async_copy(e_hbm.at[pl.ds(0, ch)], rows.at[b],
                                  r_sem.at[b]).wait()
            pltpu.make_async_copy(di_hbm.at[pl.ds(0, ch)], ivb[b],
                                  r_sem.at[b]).wait()

        def issue_scatter(c, b):
            pltpu.async_copy(rows.at[b], acc_sh.at[ivb[b]], sc_sem.at[b],
                             add=True)

        def wait_scatter(b):
            pltpu.make_async_copy(rows.at[b], acc_sh.at[pl.ds(0, ch)],
                                  sc_sem.at[b]).wait()

        for b in range(nb):
            issue_load(b, b)
        for b in range(1, nb):
            wait_load(b - 1)
            issue_scatter(b - 1, b - 1)

        def body(k, carry):
            for b in range(nb):
                c = k * nb + b
                wait_scatter(b)
                issue_load(c, b)
                b2 = (b - 1) % nb
                wait_load(b2)
                issue_scatter(c - 1, b2)
            return carry

        lax.fori_loop(1, nk, body, 0)

        wait_load(nb - 1)
        issue_scatter(nch - 1, nb - 1)
        for b in range(nb):
            wait_scatter(b)

        plsc.subcore_barrier()

        @pl.when(sid < _NS - 1)
        def _():
            pltpu.sync_copy(acc_sh.at[pl.ds(r0a, drain_a)],
                            out_hbm.at[pl.ds(cid * n + r0a, drain_a)])

        @pl.when(sid == _NS - 1)
        def _():
            pltpu.sync_copy(acc_sh.at[pl.ds(r0l, drain_last)],
                            out_hbm.at[pl.ds(cid * n + r0l, drain_last)])

    return scatter


def _edge_body(e_ref, g1_ref, g2_ref, a_ref, w2_ref, cu_ref, be2_ref,
               lns_ref, lnb_ref, enew_ref, efin_ref, esum_ref):
    i = pl.program_id(0)

    @pl.when(i == 0)
    def _():
        esum_ref[...] = jnp.zeros_like(esum_ref)

    e0 = e_ref[...].astype(F32)
    h = (
        jnp.dot(e0, a_ref[...], preferred_element_type=F32)
        + g1_ref[...].astype(F32)
        + g2_ref[...].astype(F32)
        + cu_ref[...]
    )
    h = jnp.maximum(h, 0.0)
    en = jnp.dot(h, w2_ref[...], preferred_element_type=F32) + be2_ref[...]
    enew_ref[...] = en
    esum_ref[...] += jnp.sum(en, axis=0, keepdims=True)
    efin_ref[...] = _ln(en + e0, lns_ref[...], lnb_ref[...]).astype(
        efin_ref.dtype)


def _edge_mlp(e, g1, g2, a, w2, cu, be2, lns, lnb, be, out_dtype):
    ne, d = e.shape
    grid = (ne // be,)
    blk = pl.BlockSpec((be, d), lambda i: (i, 0))
    full = pl.BlockSpec((d, d), lambda i: (0, 0))
    row = pl.BlockSpec((1, d), lambda i: (0, 0))
    return pl.pallas_call(
        _edge_body,
        grid=grid,
        in_specs=[blk, blk, blk, full, full, row, row, row, row],
        out_specs=[blk, blk, row],
        out_shape=[
            jax.ShapeDtypeStruct((ne, d), F32),
            jax.ShapeDtypeStruct((ne, d), out_dtype),
            jax.ShapeDtypeStruct((1, d), F32),
        ],
    )(e, g1, g2, a, w2, cu, be2, lns, lnb)


def _node_body(nblocks, n_nodes, n_edges,
               x_ref, p0_ref, p1_ref, u_ref, wn1_ref, bn1_ref, wn2_ref,
               bn2_ref, lnsn_ref, lnbn_ref, esum_ref, wg1_ref, bg1_ref,
               wg2_ref, bg2_ref, lnsg_ref, lnbg_ref,
               xfin_ref, ufin_ref, xsum_ref):
    i = pl.program_id(0)
    d = x_ref.shape[1]

    @pl.when(i == 0)
    def _():
        xsum_ref[...] = jnp.zeros_like(xsum_ref)
        ufin_ref[...] = jnp.zeros_like(ufin_ref)

    x0 = x_ref[...]
    agg = p0_ref[...] + p1_ref[...]
    wn1 = wn1_ref[...]
    u0 = u_ref[...]
    cu = jnp.dot(u0, wn1[2 * d:3 * d, :], preferred_element_type=F32) + bn1_ref[...]
    h = jnp.maximum(
        jnp.dot(x0, wn1[:d, :], preferred_element_type=F32)
        + jnp.dot(agg, wn1[d:2 * d, :], preferred_element_type=F32)
        + cu,
        0.0,
    )
    xn = jnp.dot(h, wn2_ref[...], preferred_element_type=F32) + bn2_ref[...]
    xsum_ref[...] += jnp.sum(xn, axis=0, keepdims=True)
    xfin_ref[...] = _ln(xn + x0, lnsn_ref[...], lnbn_ref[...])

    @pl.when(i == nblocks - 1)
    def _():
        node_mean = xsum_ref[...] * (1.0 / n_nodes)
        edge_mean = esum_ref[...] * (1.0 / n_edges)
        wg1 = wg1_ref[...]
        gi = (
            jnp.dot(u0, wg1[:d, :], preferred_element_type=F32)
            + jnp.dot(node_mean, wg1[d:2 * d, :], preferred_element_type=F32)
            + jnp.dot(edge_mean, wg1[2 * d:3 * d, :], preferred_element_type=F32)
            + bg1_ref[...]
        )
        hg = jnp.maximum(gi, 0.0)
        un = jnp.dot(hg, wg2_ref[...], preferred_element_type=F32) + bg2_ref[...]
        ufin_ref[...] = _ln(un + u0, lnsg_ref[...], lnbg_ref[...])


def _node_mlp(x, p0, p1, u, wn1, bn1, wn2, bn2, lnsn, lnbn,
              esum, wg1, bg1, wg2, bg2, lnsg, lnbg, bn, n_edges):
    n, d = x.shape
    nblocks = n // bn
    blk = pl.BlockSpec((bn, d), lambda i: (i, 0))
    full2 = pl.BlockSpec((2 * d, d), lambda i: (0, 0))
    full3 = pl.BlockSpec((3 * d, d), lambda i: (0, 0))
    fulld = pl.BlockSpec((d, d), lambda i: (0, 0))
    row = pl.BlockSpec((1, d), lambda i: (0, 0))
    body = functools.partial(_node_body, nblocks, float(n), float(n_edges))
    xfin, ufin, _ = pl.pallas_call(
        body,
        grid=(nblocks,),
        in_specs=[blk, blk, blk, row, full3, row, fulld, row, row, row,
                  row, full3, row, fulld, row, row, row],
        out_specs=[blk, row, row],
        out_shape=[
            jax.ShapeDtypeStruct((n, d), F32),
            jax.ShapeDtypeStruct((1, d), F32),
            jax.ShapeDtypeStruct((1, d), F32),
        ],
    )(x, p0, p1, u, wn1, bn1, wn2, bn2, lnsn, lnbn,
      esum, wg1, bg1, wg2, bg2, lnsg, lnbg)
    return xfin, ufin


def kernel(x, edge_attr, u, We1, be1, We2, be2, Wn1, bn1, Wn2, bn2,
           Wg1, bg1, Wg2, bg2, lns_n, lnb_n, lns_e, lnb_e, lns_g, lnb_g,
           edge_index, batch):
    n, d = x.shape
    e_cnt = edge_attr.shape[0]
    n_layers = We1.shape[0]

    src = edge_index[0]
    dst = edge_index[1]

    ch_g = 40
    ch_s = 40
    bn_node = 2000
    be_edge = 3200

    gather = _make_gather(n, d, e_cnt, ch_g)
    scatter = _make_scatter(n, d, e_cnt, ch_s)
    zeros_nd = jnp.zeros((n, d), F32)

    def row(v):
        return v.reshape(1, d)

    xc, ec, uc = x, edge_attr, u.reshape(1, d)
    for l in range(n_layers):
        we1_l = We1[l]
        ps, pd, cu = _projections(xc, we1_l, uc, row(be1[l]), bn_node)
        g1, g2 = gather(ps, pd, src, dst)
        e_new, e_fin, e_sum = _edge_mlp(
            ec, g1, g2, we1_l[:d, :], We2[l], cu, row(be2[l]),
            row(lns_e[l]), row(lnb_e[l]), be_edge,
            F32 if l == n_layers - 1 else BF16)
        parts = scatter(e_new, dst, zeros_nd)
        x_fin, u_fin = _node_mlp(
            xc, parts[:n, :], parts[n:, :], uc,
            Wn1[l], row(bn1[l]), Wn2[l], row(bn2[l]),
            row(lns_n[l]), row(lnb_n[l]),
            e_sum, Wg1[l], row(bg1[l]), Wg2[l], row(bg2[l]),
            row(lns_g[l]), row(lnb_g[l]),
            bn_node, e_cnt)
        xc, ec, uc = x_fin, e_fin, u_fin

    return xc, ec, uc.reshape(u.shape)


# speedup vs baseline: 6.7389x; 1.0567x over previous
import functools

import jax
import jax.numpy as jnp
from jax import lax
from jax.experimental import pallas as pl
from jax.experimental.pallas import tpu as pltpu
from jax.experimental.pallas import tpu_sc as plsc

F32 = jnp.float32
BF16 = jnp.bfloat16
_EPS = 1e-5

_NC = 2
_NS = 16
_NW = _NC * _NS


def _ln(v, scale, bias):
    mu = jnp.mean(v, axis=-1, keepdims=True)
    var = jnp.mean((v - mu) ** 2, axis=-1, keepdims=True)
    return (v - mu) / jnp.sqrt(var + _EPS) * scale + bias


def _proj_body(x_ref, we1_ref, u_ref, be1_ref, ps_ref, pd_ref, cu_ref):
    d = x_ref.shape[1]
    xv = x_ref[...]
    w = we1_ref[...]
    ps_ref[...] = jnp.dot(xv, w[d:2 * d, :], preferred_element_type=F32)
    pd_ref[...] = jnp.dot(xv, w[2 * d:3 * d, :], preferred_element_type=F32)

    @pl.when(pl.program_id(0) == 0)
    def _():
        cu_ref[...] = (
            jnp.dot(u_ref[...], w[3 * d:4 * d, :], preferred_element_type=F32)
            + be1_ref[...]
        )


def _projections(x, we1_l, u, be1_l, bn):
    n, d = x.shape
    grid = (n // bn,)
    return pl.pallas_call(
        _proj_body,
        grid=grid,
        in_specs=[
            pl.BlockSpec((bn, d), lambda i: (i, 0)),
            pl.BlockSpec((4 * d, d), lambda i: (0, 0)),
            pl.BlockSpec((1, d), lambda i: (0, 0)),
            pl.BlockSpec((1, d), lambda i: (0, 0)),
        ],
        out_specs=[
            pl.BlockSpec((bn, d), lambda i: (i, 0)),
            pl.BlockSpec((bn, d), lambda i: (i, 0)),
            pl.BlockSpec((1, d), lambda i: (0, 0)),
        ],
        out_shape=[
            jax.ShapeDtypeStruct((n, d), F32),
            jax.ShapeDtypeStruct((n, d), F32),
            jax.ShapeDtypeStruct((1, d), F32),
        ],
    )(x, we1_l, u, be1_l)


_NBUF = 5


def _make_gather(n, d, e, ch):
    epc = e // _NS
    nch = epc // ch
    nb = _NBUF
    nk = nch // nb
    assert nch == nb * nk
    lda = (n // _NS) & ~7
    ldl = n - lda * (_NS - 1)
    mesh = plsc.VectorSubcoreMesh(core_axis_name="c", subcore_axis_name="s")

    @functools.partial(
        pl.kernel,
        out_type=(
            jax.ShapeDtypeStruct((e, d), F32),
            jax.ShapeDtypeStruct((e, d), F32),
        ),
        mesh=mesh,
        scratch_types=(
            [pltpu.VMEM((ch,), jnp.int32) for _ in range(nb)]
            + [
                pltpu.VMEM((nb, ch, d), F32),
                pltpu.VMEM_SHARED((n, d), F32),
                pltpu.SemaphoreType.DMA((nb,)),
                pltpu.SemaphoreType.DMA((nb,)),
                pltpu.SemaphoreType.DMA((nb,)),
            ]
        ),
    )
    def gather(ps_hbm, pd_hbm, si_hbm, di_hbm, g1_hbm, g2_hbm,
               iv0, iv1, iv2, iv3, iv4, rows, tab_sh, i_sem, g_sem, w_sem):
        ivb = [iv0, iv1, iv2, iv3, iv4]
        cid = lax.axis_index("c")
        sid = lax.axis_index("s")
        base = pl.multiple_of(sid * epc, 8)
        r0a = pl.multiple_of(sid * lda, 8)
        r0l = pl.multiple_of((_NS - 1) * lda, 8)

        def load_table(tab_hbm):
            @pl.when(sid < _NS - 1)
            def _():
                pltpu.sync_copy(tab_hbm.at[pl.ds(r0a, lda)],
                                tab_sh.at[pl.ds(r0a, lda)])

            @pl.when(sid == _NS - 1)
            def _():
                pltpu.sync_copy(tab_hbm.at[pl.ds(r0l, ldl)],
                                tab_sh.at[pl.ds(r0l, ldl)])

        @pl.when(cid == 0)
        def _():
            load_table(ps_hbm)

        @pl.when(cid == 1)
        def _():
            load_table(pd_hbm)

        plsc.subcore_barrier()

        def pipeline(ix_hbm, out_hbm):
            def issue_idx(c, b):
                off = pl.multiple_of(base + c * ch, 8)
                pltpu.async_copy(ix_hbm.at[pl.ds(off, ch)], ivb[b],
                                 i_sem.at[b])

            def wait_idx(b):
                pltpu.make_async_copy(ix_hbm.at[pl.ds(0, ch)], ivb[b],
                                      i_sem.at[b]).wait()

            def issue_gather(b):
                pltpu.async_copy(tab_sh.at[ivb[b]], rows.at[b], g_sem.at[b])

            def wait_gather(b):
                pltpu.make_async_copy(tab_sh.at[pl.ds(0, ch)], rows.at[b],
                                      g_sem.at[b]).wait()

            def issue_write(c, b):
                off = pl.multiple_of(base + c * ch, 8)
                pltpu.async_copy(rows.at[b], out_hbm.at[pl.ds(off, ch)],
                                 w_sem.at[b])

            def wait_write(b):
                pltpu.make_async_copy(rows.at[b], out_hbm.at[pl.ds(0, ch)],
                                      w_sem.at[b]).wait()

            for b in range(nb):
                issue_idx(b, b)
            for b in range(nb):
                wait_idx(b)
                issue_gather(b)
                if b >= 1:
                    wait_gather(b - 1)
                    issue_write(b - 1, b - 1)
                    issue_idx(b - 1 + nb, b - 1)

            def body(k, carry):
                for b in range(nb):
                    c = k * nb + b
                    wait_write(b)
                    wait_idx(b)
                    issue_gather(b)
                    b2 = (b - 1) % nb
                    wait_gather(b2)
                    issue_write(c - 1, b2)
                    cc = c - 1 + nb

                    @pl.when(cc < nch)
                    def _():
                        issue_idx(cc, b2)
                return carry

            lax.fori_loop(1, nk, body, 0)

            wait_gather(nb - 1)
            issue_write(nch - 1, nb - 1)
            for b in range(nb):
                wait_write(b)

        @pl.when(cid == 0)
        def _():
            pipeline(si_hbm, g1_hbm)

        @pl.when(cid == 1)
        def _():
            pipeline(di_hbm, g2_hbm)

    return gather


def _make_scatter(n, d, e, ch):
    epw = e // _NW
    nch = epw // ch
    nb = _NBUF
    nk = nch // nb
    assert nch == nb * nk
    drain_a = (n // _NS) & ~7
    drain_last = n - drain_a * (_NS - 1)
    mesh = plsc.VectorSubcoreMesh(core_axis_name="c", subcore_axis_name="s")

    @functools.partial(
        pl.kernel,
        out_type=jax.ShapeDtypeStruct((_NC * n, d), F32),
        mesh=mesh,
        scratch_types=(
            [pltpu.VMEM((ch,), jnp.int32) for _ in range(nb)]
            + [
                pltpu.VMEM((nb, ch, d), F32),
                pltpu.VMEM_SHARED((n, d), F32),
                pltpu.SemaphoreType.DMA((nb,)),
                pltpu.SemaphoreType.DMA((nb,)),
            ]
        ),
    )
    def scatter(e_hbm, di_hbm, zer_hbm, out_hbm, iv0, iv1, iv2, iv3, iv4,
                rows, acc_sh, r_sem, sc_sem):
        ivb = [iv0, iv1, iv2, iv3, iv4]
        cid = lax.axis_index("c")
        sid = lax.axis_index("s")
        wid = cid * _NS + sid
        base = pl.multiple_of(wid * epw, 8)

        r0a = pl.multiple_of(sid * drain_a, 8)
        r0l = pl.multiple_of((_NS - 1) * drain_a, 8)

        @pl.when(sid < _NS - 1)
        def _():
            pltpu.sync_copy(zer_hbm.at[pl.ds(r0a, drain_a)],
                            acc_sh.at[pl.ds(r0a, drain_a)])

        @pl.when(sid == _NS - 1)
        def _():
            pltpu.sync_copy(zer_hbm.at[pl.ds(r0l, drain_last)],
                            acc_sh.at[pl.ds(r0l, drain_last)])

        plsc.subcore_barrier()

        def issue_load(c, b):
            off = pl.multiple_of(base + c * ch, 8)
            pltpu.async_copy(e_hbm.at[pl.ds(off, ch)], rows.at[b],
                             r_sem.at[b])
            pltpu.async_copy(di_hbm.at[pl.ds(off, ch)], ivb[b], r_sem.at[b])

        def wait_load(b):
            pltpu.make_async_copy(e_hbm.at[pl.ds(0, ch)], rows.at[b],
                                  r_sem.at[b]).wait()
            pltpu.make_async_copy(di_hbm.at[pl.ds(0, ch)], ivb[b],
                                  r_sem.at[b]).wait()

        def issue_scatter(c, b):
            pltpu.async_copy(rows.at[b], acc_sh.at[ivb[b]], sc_sem.at[b],
                             add=True)

        def wait_scatter(b):
            pltpu.make_async_copy(rows.at[b], acc_sh.at[pl.ds(0, ch)],
                                  sc_sem.at[b]).wait()

        for b in range(nb):
            issue_load(b, b)
        for b in range(1, nb):
            wait_load(b - 1)
            issue_scatter(b - 1, b - 1)

        def body(k, carry):
            for b in range(nb):
                c = k * nb + b
                wait_scatter(b)
                issue_load(c, b)
                b2 = (b - 1) % nb
                wait_load(b2)
                issue_scatter(c - 1, b2)
            return carry

        lax.fori_loop(1, nk, body, 0)

        wait_load(nb - 1)
        issue_scatter(nch - 1, nb - 1)
        for b in range(nb):
            wait_scatter(b)

        plsc.subcore_barrier()

        @pl.when(sid < _NS - 1)
        def _():
            pltpu.sync_copy(acc_sh.at[pl.ds(r0a, drain_a)],
                            out_hbm.at[pl.ds(cid * n + r0a, drain_a)])

        @pl.when(sid == _NS - 1)
        def _():
            pltpu.sync_copy(acc_sh.at[pl.ds(r0l, drain_last)],
                            out_hbm.at[pl.ds(cid * n + r0l, drain_last)])

    return scatter


def _edge_body(e_ref, g1_ref, g2_ref, a_ref, w2_ref, cu_ref, be2_ref,
               lns_ref, lnb_ref, enew_ref, efin_ref, esum_ref):
    i = pl.program_id(0)

    @pl.when(i == 0)
    def _():
        esum_ref[...] = jnp.zeros_like(esum_ref)

    e0 = e_ref[...].astype(F32)
    h = (
        jnp.dot(e0, a_ref[...], preferred_element_type=F32)
        + g1_ref[...].astype(F32)
        + g2_ref[...].astype(F32)
        + cu_ref[...]
    )
    h = jnp.maximum(h, 0.0)
    en = jnp.dot(h, w2_ref[...], preferred_element_type=F32) + be2_ref[...]
    enew_ref[...] = en
    esum_ref[...] += jnp.sum(en, axis=0, keepdims=True)
    efin_ref[...] = _ln(en + e0, lns_ref[...], lnb_ref[...]).astype(
        efin_ref.dtype)


def _edge_mlp(e, g1, g2, a, w2, cu, be2, lns, lnb, be, out_dtype):
    ne, d = e.shape
    grid = (ne // be,)
    blk = pl.BlockSpec((be, d), lambda i: (i, 0))
    full = pl.BlockSpec((d, d), lambda i: (0, 0))
    row = pl.BlockSpec((1, d), lambda i: (0, 0))
    return pl.pallas_call(
        _edge_body,
        grid=grid,
        in_specs=[blk, blk, blk, full, full, row, row, row, row],
        out_specs=[blk, blk, row],
        out_shape=[
            jax.ShapeDtypeStruct((ne, d), F32),
            jax.ShapeDtypeStruct((ne, d), out_dtype),
            jax.ShapeDtypeStruct((1, d), F32),
        ],
    )(e, g1, g2, a, w2, cu, be2, lns, lnb)


def _node_body(nblocks, n_nodes, n_edges,
               x_ref, p0_ref, p1_ref, u_ref, wn1_ref, bn1_ref, wn2_ref,
               bn2_ref, lnsn_ref, lnbn_ref, esum_ref, wg1_ref, bg1_ref,
               wg2_ref, bg2_ref, lnsg_ref, lnbg_ref,
               xfin_ref, ufin_ref, xsum_ref):
    i = pl.program_id(0)
    d = x_ref.shape[1]

    @pl.when(i == 0)
    def _():
        xsum_ref[...] = jnp.zeros_like(xsum_ref)
        ufin_ref[...] = jnp.zeros_like(ufin_ref)

    x0 = x_ref[...]
    agg = p0_ref[...] + p1_ref[...]
    wn1 = wn1_ref[...]
    u0 = u_ref[...]
    cu = jnp.dot(u0, wn1[2 * d:3 * d, :], preferred_element_type=F32) + bn1_ref[...]
    h = jnp.maximum(
        jnp.dot(x0, wn1[:d, :], preferred_element_type=F32)
        + jnp.dot(agg, wn1[d:2 * d, :], preferred_element_type=F32)
        + cu,
        0.0,
    )
    xn = jnp.dot(h, wn2_ref[...], preferred_element_type=F32) + bn2_ref[...]
    xsum_ref[...] += jnp.sum(xn, axis=0, keepdims=True)
    xfin_ref[...] = _ln(xn + x0, lnsn_ref[...], lnbn_ref[...])

    @pl.when(i == nblocks - 1)
    def _():
        node_mean = xsum_ref[...] * (1.0 / n_nodes)
        edge_mean = esum_ref[...] * (1.0 / n_edges)
        wg1 = wg1_ref[...]
        gi = (
            jnp.dot(u0, wg1[:d, :], preferred_element_type=F32)
            + jnp.dot(node_mean, wg1[d:2 * d, :], preferred_element_type=F32)
            + jnp.dot(edge_mean, wg1[2 * d:3 * d, :], preferred_element_type=F32)
            + bg1_ref[...]
        )
        hg = jnp.maximum(gi, 0.0)
        un = jnp.dot(hg, wg2_ref[...], preferred_element_type=F32) + bg2_ref[...]
        ufin_ref[...] = _ln(un + u0, lnsg_ref[...], lnbg_ref[...])


def _node_mlp(x, p0, p1, u, wn1, bn1, wn2, bn2, lnsn, lnbn,
              esum, wg1, bg1, wg2, bg2, lnsg, lnbg, bn, n_edges):
    n, d = x.shape
    nblocks = n // bn
    blk = pl.BlockSpec((bn, d), lambda i: (i, 0))
    full2 = pl.BlockSpec((2 * d, d), lambda i: (0, 0))
    full3 = pl.BlockSpec((3 * d, d), lambda i: (0, 0))
    fulld = pl.BlockSpec((d, d), lambda i: (0, 0))
    row = pl.BlockSpec((1, d), lambda i: (0, 0))
    body = functools.partial(_node_body, nblocks, float(n), float(n_edges))
    xfin, ufin, _ = pl.pallas_call(
        body,
        grid=(nblocks,),
        in_specs=[blk, blk, blk, row, full3, row, fulld, row, row, row,
                  row, full3, row, fulld, row, row, row],
        out_specs=[blk, row, row],
        out_shape=[
            jax.ShapeDtypeStruct((n, d), F32),
            jax.ShapeDtypeStruct((1, d), F32),
            jax.ShapeDtypeStruct((1, d), F32),
        ],
    )(x, p0, p1, u, wn1, bn1, wn2, bn2, lnsn, lnbn,
      esum, wg1, bg1, wg2, bg2, lnsg, lnbg)
    return xfin, ufin


def kernel(x, edge_attr, u, We1, be1, We2, be2, Wn1, bn1, Wn2, bn2,
           Wg1, bg1, Wg2, bg2, lns_n, lnb_n, lns_e, lnb_e, lns_g, lnb_g,
           edge_index, batch):
    n, d = x.shape
    e_cnt = edge_attr.shape[0]
    n_layers = We1.shape[0]

    src = edge_index[0]
    dst = edge_index[1]

    ch_g = 40
    ch_s = 40
    bn_node = 2000
    be_edge = 3200

    gather = _make_gather(n, d, e_cnt, ch_g)
    scatter = _make_scatter(n, d, e_cnt, ch_s)
    zeros_nd = jnp.zeros((n, d), F32)

    def row(v):
        return v.reshape(1, d)

    xc, ec, uc = x, edge_attr, u.reshape(1, d)
    for l in range(n_layers):
        we1_l = We1[l]
        ps, pd, cu = _projections(xc, we1_l, uc, row(be1[l]), bn_node)
        g1, g2 = gather(ps, pd, src, dst)
        e_new, e_fin, e_sum = _edge_mlp(
            ec, g1, g2, we1_l[:d, :], We2[l], cu, row(be2[l]),
            row(lns_e[l]), row(lnb_e[l]), be_edge,
            F32 if l == n_layers - 1 else BF16)
        parts = scatter(e_new, dst, zeros_nd)
        x_fin, u_fin = _node_mlp(
            xc, parts[:n, :], parts[n:, :], uc,
            Wn1[l], row(bn1[l]), Wn2[l], row(bn2[l]),
            row(lns_n[l]), row(lnb_n[l]),
            e_sum, Wg1[l], row(bg1[l]), Wg2[l], row(bg2[l]),
            row(lns_g[l]), row(lnb_g[l]),
            bn_node, e_cnt)
        xc, ec, uc = x_fin, e_fin, u_fin

    return xc, ec, uc.reshape(u.shape)


# speedup vs baseline: 7.3503x; 1.0907x over previous
import functools

import jax
import jax.numpy as jnp
from jax import lax
from jax.experimental import pallas as pl
from jax.experimental.pallas import tpu as pltpu
from jax.experimental.pallas import tpu_sc as plsc

F32 = jnp.float32
BF16 = jnp.bfloat16
_EPS = 1e-5

_NC = 2
_NS = 16
_NW = _NC * _NS


def _ln(v, scale, bias):
    mu = jnp.mean(v, axis=-1, keepdims=True)
    var = jnp.mean((v - mu) ** 2, axis=-1, keepdims=True)
    return (v - mu) / jnp.sqrt(var + _EPS) * scale + bias


def _proj_body(x_ref, we1_ref, u_ref, be1_ref, ps_ref, pd_ref, cu_ref):
    d = x_ref.shape[1]
    xv = x_ref[...]
    w = we1_ref[...]
    ps_ref[...] = jnp.dot(xv, w[d:2 * d, :], preferred_element_type=F32)
    pd_ref[...] = jnp.dot(xv, w[2 * d:3 * d, :], preferred_element_type=F32)

    @pl.when(pl.program_id(0) == 0)
    def _():
        cu_ref[...] = (
            jnp.dot(u_ref[...], w[3 * d:4 * d, :], preferred_element_type=F32)
            + be1_ref[...]
        )


def _projections(x, we1_l, u, be1_l, bn):
    n, d = x.shape
    grid = (n // bn,)
    return pl.pallas_call(
        _proj_body,
        grid=grid,
        in_specs=[
            pl.BlockSpec((bn, d), lambda i: (i, 0)),
            pl.BlockSpec((4 * d, d), lambda i: (0, 0)),
            pl.BlockSpec((1, d), lambda i: (0, 0)),
            pl.BlockSpec((1, d), lambda i: (0, 0)),
        ],
        out_specs=[
            pl.BlockSpec((bn, d), lambda i: (i, 0)),
            pl.BlockSpec((bn, d), lambda i: (i, 0)),
            pl.BlockSpec((1, d), lambda i: (0, 0)),
        ],
        out_shape=[
            jax.ShapeDtypeStruct((n, d), F32),
            jax.ShapeDtypeStruct((n, d), F32),
            jax.ShapeDtypeStruct((1, d), F32),
        ],
    )(x, we1_l, u, be1_l)


_NBUF = 5


def _make_gather(n, d, e_off, es, ch):
    epc = es // _NS
    nch = epc // ch
    nb = _NBUF
    nk = nch // nb
    assert nch == nb * nk
    lda = (n // _NS) & ~7
    ldl = n - lda * (_NS - 1)
    mesh = plsc.VectorSubcoreMesh(core_axis_name="c", subcore_axis_name="s")

    @functools.partial(
        pl.kernel,
        out_type=(
            jax.ShapeDtypeStruct((es, d), F32),
            jax.ShapeDtypeStruct((es, d), F32),
        ),
        mesh=mesh,
        scratch_types=(
            [pltpu.VMEM((ch,), jnp.int32) for _ in range(nb)]
            + [
                pltpu.VMEM((nb, ch, d), F32),
                pltpu.VMEM_SHARED((n, d), F32),
                pltpu.SemaphoreType.DMA((nb,)),
                pltpu.SemaphoreType.DMA((nb,)),
                pltpu.SemaphoreType.DMA((nb,)),
            ]
        ),
    )
    def gather(ps_hbm, pd_hbm, si_hbm, di_hbm, g1_hbm, g2_hbm,
               iv0, iv1, iv2, iv3, iv4, rows, tab_sh, i_sem, g_sem, w_sem):
        ivb = [iv0, iv1, iv2, iv3, iv4]
        cid = lax.axis_index("c")
        sid = lax.axis_index("s")
        base = pl.multiple_of(sid * epc, 8)
        ibase = pl.multiple_of(e_off + sid * epc, 8)
        r0a = pl.multiple_of(sid * lda, 8)
        r0l = pl.multiple_of((_NS - 1) * lda, 8)

        def load_table(tab_hbm):
            @pl.when(sid < _NS - 1)
            def _():
                pltpu.sync_copy(tab_hbm.at[pl.ds(r0a, lda)],
                                tab_sh.at[pl.ds(r0a, lda)])

            @pl.when(sid == _NS - 1)
            def _():
                pltpu.sync_copy(tab_hbm.at[pl.ds(r0l, ldl)],
                                tab_sh.at[pl.ds(r0l, ldl)])

        @pl.when(cid == 0)
        def _():
            load_table(ps_hbm)

        @pl.when(cid == 1)
        def _():
            load_table(pd_hbm)

        plsc.subcore_barrier()

        def pipeline(ix_hbm, out_hbm):
            def issue_idx(c, b):
                off = pl.multiple_of(ibase + c * ch, 8)
                pltpu.async_copy(ix_hbm.at[pl.ds(off, ch)], ivb[b],
                                 i_sem.at[b])

            def wait_idx(b):
                pltpu.make_async_copy(ix_hbm.at[pl.ds(0, ch)], ivb[b],
                                      i_sem.at[b]).wait()

            def issue_gather(b):
                pltpu.async_copy(tab_sh.at[ivb[b]], rows.at[b], g_sem.at[b])

            def wait_gather(b):
                pltpu.make_async_copy(tab_sh.at[pl.ds(0, ch)], rows.at[b],
                                      g_sem.at[b]).wait()

            def issue_write(c, b):
                off = pl.multiple_of(base + c * ch, 8)
                pltpu.async_copy(rows.at[b], out_hbm.at[pl.ds(off, ch)],
                                 w_sem.at[b])

            def wait_write(b):
                pltpu.make_async_copy(rows.at[b], out_hbm.at[pl.ds(0, ch)],
                                      w_sem.at[b]).wait()

            for b in range(nb):
                issue_idx(b, b)
            for b in range(nb):
                wait_idx(b)
                issue_gather(b)
                if b >= 1:
                    wait_gather(b - 1)
                    issue_write(b - 1, b - 1)
                    issue_idx(b - 1 + nb, b - 1)

            def body(k, carry):
                for b in range(nb):
                    c = k * nb + b
                    wait_write(b)
                    wait_idx(b)
                    issue_gather(b)
                    b2 = (b - 1) % nb
                    wait_gather(b2)
                    issue_write(c - 1, b2)
                    cc = c - 1 + nb

                    @pl.when(cc < nch)
                    def _():
                        issue_idx(cc, b2)
                return carry

            lax.fori_loop(1, nk, body, 0)

            wait_gather(nb - 1)
            issue_write(nch - 1, nb - 1)
            for b in range(nb):
                wait_write(b)

        @pl.when(cid == 0)
        def _():
            pipeline(si_hbm, g1_hbm)

        @pl.when(cid == 1)
        def _():
            pipeline(di_hbm, g2_hbm)

    return gather


def _make_scatter(n, d, e_off, es, ch):
    epw = es // _NW
    nch = epw // ch
    nb = _NBUF
    nk = nch // nb
    assert nch == nb * nk
    drain_a = (n // _NS) & ~7
    drain_last = n - drain_a * (_NS - 1)
    mesh = plsc.VectorSubcoreMesh(core_axis_name="c", subcore_axis_name="s")

    @functools.partial(
        pl.kernel,
        out_type=jax.ShapeDtypeStruct((_NC * n, d), F32),
        mesh=mesh,
        scratch_types=(
            [pltpu.VMEM((ch,), jnp.int32) for _ in range(nb)]
            + [
                pltpu.VMEM((nb, ch, d), F32),
                pltpu.VMEM_SHARED((n, d), F32),
                pltpu.SemaphoreType.DMA((nb,)),
                pltpu.SemaphoreType.DMA((nb,)),
            ]
        ),
    )
    def scatter(e_hbm, di_hbm, zer_hbm, out_hbm, iv0, iv1, iv2, iv3, iv4,
                rows, acc_sh, r_sem, sc_sem):
        ivb = [iv0, iv1, iv2, iv3, iv4]
        cid = lax.axis_index("c")
        sid = lax.axis_index("s")
        wid = cid * _NS + sid
        base = pl.multiple_of(wid * epw, 8)
        ibase = pl.multiple_of(e_off + wid * epw, 8)

        r0a = pl.multiple_of(sid * drain_a, 8)
        r0l = pl.multiple_of((_NS - 1) * drain_a, 8)

        @pl.when(sid < _NS - 1)
        def _():
            pltpu.sync_copy(zer_hbm.at[pl.ds(r0a, drain_a)],
                            acc_sh.at[pl.ds(r0a, drain_a)])

        @pl.when(sid == _NS - 1)
        def _():
            pltpu.sync_copy(zer_hbm.at[pl.ds(r0l, drain_last)],
                            acc_sh.at[pl.ds(r0l, drain_last)])

        plsc.subcore_barrier()

        def issue_load(c, b):
            off = pl.multiple_of(base + c * ch, 8)
            ioff = pl.multiple_of(ibase + c * ch, 8)
            pltpu.async_copy(e_hbm.at[pl.ds(off, ch)], rows.at[b],
                             r_sem.at[b])
            pltpu.async_copy(di_hbm.at[pl.ds(ioff, ch)], ivb[b], r_sem.at[b])

        def wait_load(b):
            pltpu.make_async_copy(e_hbm.at[pl.ds(0, ch)], rows.at[b],
                                  r_sem.at[b]).wait()
            pltpu.make_async_copy(di_hbm.at[pl.ds(0, ch)], ivb[b],
                                  r_sem.at[b]).wait()

        def issue_scatter(c, b):
            pltpu.async_copy(rows.at[b], acc_sh.at[ivb[b]], sc_sem.at[b],
                             add=True)

        def wait_scatter(b):
            pltpu.make_async_copy(rows.at[b], acc_sh.at[pl.ds(0, ch)],
                                  sc_sem.at[b]).wait()

        for b in range(nb):
            issue_load(b, b)
        for b in range(1, nb):
            wait_load(b - 1)
            issue_scatter(b - 1, b - 1)

        def body(k, carry):
            for b in range(nb):
                c = k * nb + b
                wait_scatter(b)
                issue_load(c, b)
                b2 = (b - 1) % nb
                wait_load(b2)
                issue_scatter(c - 1, b2)
            return carry

        lax.fori_loop(1, nk, body, 0)

        wait_load(nb - 1)
        issue_scatter(nch - 1, nb - 1)
        for b in range(nb):
            wait_scatter(b)

        plsc.subcore_barrier()

        @pl.when(sid < _NS - 1)
        def _():
            pltpu.sync_copy(acc_sh.at[pl.ds(r0a, drain_a)],
                            out_hbm.at[pl.ds(cid * n + r0a, drain_a)])

        @pl.when(sid == _NS - 1)
        def _():
            pltpu.sync_copy(acc_sh.at[pl.ds(r0l, drain_last)],
                            out_hbm.at[pl.ds(cid * n + r0l, drain_last)])

    return scatter


def _edge_body(e_ref, g1_ref, g2_ref, a_ref, w2_ref, cu_ref, be2_ref,
               lns_ref, lnb_ref, *rest):
    enew_ref, efin_ref, esum_ref = rest[-3:]
    i = pl.program_id(0)

    @pl.when(i == 0)
    def _():
        esum_ref[...] = jnp.zeros_like(esum_ref)

    e0 = e_ref[...].astype(F32)
    h = (
        jnp.dot(e0, a_ref[...], preferred_element_type=F32)
        + g1_ref[...].astype(F32)
        + g2_ref[...].astype(F32)
        + cu_ref[...]
    )
    h = jnp.maximum(h, 0.0)
    en = jnp.dot(h, w2_ref[...], preferred_element_type=F32) + be2_ref[...]
    enew_ref[...] = en
    esum_ref[...] += jnp.sum(en, axis=0, keepdims=True)
    efin_ref[...] = _ln(en + e0, lns_ref[...], lnb_ref[...]).astype(
        efin_ref.dtype)


def _edge_mlp(e, g1, g2, a, w2, cu, be2, lns, lnb, be, out_dtype,
              blk_off, carrier):
    ne_full, d = e.shape
    es = g1.shape[0]
    grid = (es // be,)
    eblk = pl.BlockSpec((be, d), lambda i: (i + blk_off, 0))
    sblk = pl.BlockSpec((be, d), lambda i: (i, 0))
    full = pl.BlockSpec((d, d), lambda i: (0, 0))
    row = pl.BlockSpec((1, d), lambda i: (0, 0))
    in_specs = [eblk, sblk, sblk, full, full, row, row, row, row]
    args = [e, g1, g2, a, w2, cu, be2, lns, lnb]
    aliases = {}
    if carrier is not None:
        in_specs.append(pl.BlockSpec((8, d), lambda i: (0, 0)))
        args.append(carrier)
        aliases = {9: 1}
    return pl.pallas_call(
        _edge_body,
        grid=grid,
        in_specs=in_specs,
        out_specs=[sblk, eblk, row],
        out_shape=[
            jax.ShapeDtypeStruct((es, d), F32),
            jax.ShapeDtypeStruct((ne_full, d), out_dtype),
            jax.ShapeDtypeStruct((1, d), F32),
        ],
        input_output_aliases=aliases,
    )(*args)


def _node_body(nblocks, n_nodes, n_edges,
               x_ref, p0_ref, p1_ref, p2_ref, p3_ref, u_ref, wn1_ref,
               bn1_ref, wn2_ref, bn2_ref, lnsn_ref, lnbn_ref, esum0_ref,
               esum1_ref, wg1_ref, bg1_ref, wg2_ref, bg2_ref, lnsg_ref,
               lnbg_ref, xfin_ref, ufin_ref, xsum_ref):
    i = pl.program_id(0)
    d = x_ref.shape[1]

    @pl.when(i == 0)
    def _():
        xsum_ref[...] = jnp.zeros_like(xsum_ref)
        ufin_ref[...] = jnp.zeros_like(ufin_ref)

    x0 = x_ref[...]
    agg = (p0_ref[...] + p1_ref[...]) + (p2_ref[...] + p3_ref[...])
    wn1 = wn1_ref[...]
    u0 = u_ref[...]
    cu = jnp.dot(u0, wn1[2 * d:3 * d, :], preferred_element_type=F32) + bn1_ref[...]
    h = jnp.maximum(
        jnp.dot(x0, wn1[:d, :], preferred_element_type=F32)
        + jnp.dot(agg, wn1[d:2 * d, :], preferred_element_type=F32)
        + cu,
        0.0,
    )
    xn = jnp.dot(h, wn2_ref[...], preferred_element_type=F32) + bn2_ref[...]
    xsum_ref[...] += jnp.sum(xn, axis=0, keepdims=True)
    xfin_ref[...] = _ln(xn + x0, lnsn_ref[...], lnbn_ref[...])

    @pl.when(i == nblocks - 1)
    def _():
        node_mean = xsum_ref[...] * (1.0 / n_nodes)
        edge_mean = (esum0_ref[...] + esum1_ref[...]) * (1.0 / n_edges)
        wg1 = wg1_ref[...]
        gi = (
            jnp.dot(u0, wg1[:d, :], preferred_element_type=F32)
            + jnp.dot(node_mean, wg1[d:2 * d, :], preferred_element_type=F32)
            + jnp.dot(edge_mean, wg1[2 * d:3 * d, :], preferred_element_type=F32)
            + bg1_ref[...]
        )
        hg = jnp.maximum(gi, 0.0)
        un = jnp.dot(hg, wg2_ref[...], preferred_element_type=F32) + bg2_ref[...]
        ufin_ref[...] = _ln(un + u0, lnsg_ref[...], lnbg_ref[...])


def _node_mlp(x, parts0, parts1, u, wn1, bn1, wn2, bn2, lnsn, lnbn,
              esum0, esum1, wg1, bg1, wg2, bg2, lnsg, lnbg, bn, n_edges):
    n, d = x.shape
    nblocks = n // bn
    blk = pl.BlockSpec((bn, d), lambda i: (i, 0))
    blk_hi = pl.BlockSpec((bn, d), lambda i: (i + nblocks, 0))
    full3 = pl.BlockSpec((3 * d, d), lambda i: (0, 0))
    fulld = pl.BlockSpec((d, d), lambda i: (0, 0))
    row = pl.BlockSpec((1, d), lambda i: (0, 0))
    body = functools.partial(_node_body, nblocks, float(n), float(n_edges))
    xfin, ufin, _ = pl.pallas_call(
        body,
        grid=(nblocks,),
        in_specs=[blk, blk, blk_hi, blk, blk_hi, row, full3, row, fulld,
                  row, row, row, row, row, full3, row, fulld, row, row, row],
        out_specs=[blk, row, row],
        out_shape=[
            jax.ShapeDtypeStruct((n, d), F32),
            jax.ShapeDtypeStruct((1, d), F32),
            jax.ShapeDtypeStruct((1, d), F32),
        ],
    )(x, parts0, parts0, parts1, parts1, u, wn1, bn1, wn2, bn2, lnsn, lnbn,
      esum0, esum1, wg1, bg1, wg2, bg2, lnsg, lnbg)
    return xfin, ufin


def kernel(x, edge_attr, u, We1, be1, We2, be2, Wn1, bn1, Wn2, bn2,
           Wg1, bg1, Wg2, bg2, lns_n, lnb_n, lns_e, lnb_e, lns_g, lnb_g,
           edge_index, batch):
    n, d = x.shape
    e_cnt = edge_attr.shape[0]
    n_layers = We1.shape[0]

    src = edge_index[0]
    dst = edge_index[1]

    ch_g = 40
    ch_s = 40
    bn_node = 2000
    be_edge = 3200
    n_slices = 2
    es = e_cnt // n_slices

    gathers = [_make_gather(n, d, s * es, es, ch_g) for s in range(n_slices)]
    scatters = [_make_scatter(n, d, s * es, es, ch_s)
                for s in range(n_slices)]
    zeros_nd = jnp.zeros((n, d), F32)

    def row(v):
        return v.reshape(1, d)

    xc, ec, uc = x, edge_attr, u.reshape(1, d)
    for l in range(n_layers):
        we1_l = We1[l]
        out_dt = F32 if l == n_layers - 1 else BF16
        ps, pd, cu = _projections(xc, we1_l, uc, row(be1[l]), bn_node)
        e_fin = None
        e_news, e_sums = [], []
        for s in range(n_slices):
            g1, g2 = gathers[s](ps, pd, src, dst)
            e_new, e_fin, e_sum = _edge_mlp(
                ec, g1, g2, we1_l[:d, :], We2[l], cu, row(be2[l]),
                row(lns_e[l]), row(lnb_e[l]), be_edge, out_dt,
                s * (es // be_edge), e_fin)
            e_news.append(e_new)
            e_sums.append(e_sum)
        parts = [scatters[s](e_news[s], dst, zeros_nd)
                 for s in range(n_slices)]
        x_fin, u_fin = _node_mlp(
            xc, parts[0], parts[1], uc,
            Wn1[l], row(bn1[l]), Wn2[l], row(bn2[l]),
            row(lns_n[l]), row(lnb_n[l]),
            e_sums[0], e_sums[1], Wg1[l], row(bg1[l]), Wg2[l], row(bg2[l]),
            row(lns_g[l]), row(lnb_g[l]),
            bn_node, e_cnt)
        xc, ec, uc = x_fin, e_fin, u_fin

    return xc, ec, uc.reshape(u.shape)
